# baseline (speedup 1.0000x reference)
import jax
import jax.numpy as jnp
from jax.experimental import pallas as pl
from jax.experimental.pallas import tpu as pltpu

M = 100000
D = 256
BLK = 5000
NB = M // BLK
K = 16
NEG_BIG = -1e9
NEG_INF = -3.0e38


def _recall_kernel(q_ref, mem_ref, act_ref, out_ref, scores_s, sims_s):
    i = pl.program_id(0)
    q = q_ref[...]
    qn = q / (jnp.sqrt(jnp.sum(q * q)) + 1e-8)
    x = mem_ref[...]
    ones = jnp.ones((1, D), jnp.float32)
    sumsq = jax.lax.dot_general(
        ones, x * x, (((1,), (1,)), ((), ())),
        preferred_element_type=jnp.float32)
    dotq = jax.lax.dot_general(
        qn, x, (((1,), (1,)), ((), ())),
        preferred_element_type=jnp.float32)
    sims = dotq / (jnp.sqrt(sumsq) + 1e-8)
    act = act_ref[0]
    scores = jnp.where(sims > 0.0, act, NEG_BIG)
    scores_s[pl.ds(i, 1), :] = scores
    sims_s[pl.ds(i, 1), :] = sims

    @pl.when(i == NB - 1)
    def _select():
        sc = scores_s[...]
        sm = sims_s[...]
        row = jax.lax.broadcasted_iota(jnp.int32, (NB, BLK), 0)
        col = jax.lax.broadcasted_iota(jnp.int32, (NB, BLK), 1)
        gidx = row * BLK + col
        lane = jax.lax.broadcasted_iota(jnp.int32, (1, K), 1)
        out0 = jnp.zeros((1, K), jnp.float32)
        out1 = jnp.zeros((1, K), jnp.float32)
        for k in range(K):
            m = jnp.max(sc)
            idx = jnp.min(jnp.where(sc == m, gidx, jnp.int32(2**31 - 1)))
            sel = gidx == idx
            simv = jnp.max(jnp.where(sel, sm, NEG_INF))
            out0 = jnp.where(lane == k, m, out0)
            out1 = jnp.where(lane == k, simv, out1)
            sc = jnp.where(sel, NEG_INF, sc)
        out_ref[0:1, :] = out0
        out_ref[1:2, :] = out1


def kernel(query_features, mem_features, activation_strength, topk):
    q = query_features.reshape(1, D)
    act = activation_strength.reshape(NB, 1, BLK)
    out = pl.pallas_call(
        _recall_kernel,
        grid=(NB,),
        in_specs=[
            pl.BlockSpec((1, D), lambda i: (0, 0)),
            pl.BlockSpec((BLK, D), lambda i: (i, 0)),
            pl.BlockSpec((1, 1, BLK), lambda i: (i, 0, 0)),
        ],
        out_specs=pl.BlockSpec((2, K), lambda i: (0, 0)),
        out_shape=jax.ShapeDtypeStruct((2, K), jnp.float32),
        scratch_shapes=[
            pltpu.VMEM((NB, BLK), jnp.float32),
            pltpu.VMEM((NB, BLK), jnp.float32),
        ],
        compiler_params=pltpu.CompilerParams(
            dimension_semantics=("arbitrary",)),
    )(q, mem_features, act)
    toff = (jnp.asarray(topk) - K).astype(jnp.float32)
    return out.at[0, :].add(toff)


# speedup vs baseline: 1.0367x; 1.0367x over previous
import jax
import jax.numpy as jnp
from jax.experimental import pallas as pl
from jax.experimental.pallas import tpu as pltpu

M = 100000
D = 256
BLK = 10000
NB = M // BLK
K = 16
NEG_BIG = -1e9
NEG_INF = -3.0e38


def _recall_kernel(q_ref, mem_ref, act_ref, out_ref, scores_s, sims_s):
    i = pl.program_id(0)
    q = q_ref[...]
    qn = q / (jnp.sqrt(jnp.sum(q * q)) + 1e-8)
    x = mem_ref[...]
    ones = jnp.ones((1, D), jnp.float32)
    sumsq = jax.lax.dot_general(
        ones, x * x, (((1,), (1,)), ((), ())),
        preferred_element_type=jnp.float32)
    dotq = jax.lax.dot_general(
        qn, x, (((1,), (1,)), ((), ())),
        preferred_element_type=jnp.float32)
    sims = dotq / (jnp.sqrt(sumsq) + 1e-8)
    act = act_ref[0]
    scores = jnp.where(sims > 0.0, act, NEG_BIG)
    scores_s[pl.ds(i, 1), :] = scores
    sims_s[pl.ds(i, 1), :] = sims

    @pl.when(i == NB - 1)
    def _select():
        sc = scores_s[...]
        sm = sims_s[...]
        row = jax.lax.broadcasted_iota(jnp.int32, (NB, BLK), 0)
        col = jax.lax.broadcasted_iota(jnp.int32, (NB, BLK), 1)
        gidx = row * BLK + col
        lane = jax.lax.broadcasted_iota(jnp.int32, (1, K), 1)
        out0 = jnp.zeros((1, K), jnp.float32)
        out1 = jnp.zeros((1, K), jnp.float32)
        for k in range(K):
            m = jnp.max(sc)
            idx = jnp.min(jnp.where(sc == m, gidx, jnp.int32(2**31 - 1)))
            sel = gidx == idx
            simv = jnp.max(jnp.where(sel, sm, NEG_INF))
            out0 = jnp.where(lane == k, m, out0)
            out1 = jnp.where(lane == k, simv, out1)
            sc = jnp.where(sel, NEG_INF, sc)
        out_ref[0:1, :] = out0
        out_ref[1:2, :] = out1


def kernel(query_features, mem_features, activation_strength, topk):
    q = query_features.reshape(1, D)
    act = activation_strength.reshape(NB, 1, BLK)
    out = pl.pallas_call(
        _recall_kernel,
        grid=(NB,),
        in_specs=[
            pl.BlockSpec((1, D), lambda i: (0, 0)),
            pl.BlockSpec((BLK, D), lambda i: (i, 0)),
            pl.BlockSpec((1, 1, BLK), lambda i: (i, 0, 0)),
        ],
        out_specs=pl.BlockSpec((2, K), lambda i: (0, 0)),
        out_shape=jax.ShapeDtypeStruct((2, K), jnp.float32),
        scratch_shapes=[
            pltpu.VMEM((NB, BLK), jnp.float32),
            pltpu.VMEM((NB, BLK), jnp.float32),
        ],
        compiler_params=pltpu.CompilerParams(
            dimension_semantics=("arbitrary",)),
    )(q, mem_features, act)
    toff = (jnp.asarray(topk) - K).astype(jnp.float32)
    return out.at[0, :].add(toff)


# speedup vs baseline: 1.0750x; 1.0370x over previous
import jax
import jax.numpy as jnp
from jax.experimental import pallas as pl
from jax.experimental.pallas import tpu as pltpu

M = 100000
D = 256
SUB = 20
LN = 512
BLKP = SUB * LN
NB = 10
K = 16
NEG_BIG = -1e9
NEG_INF = -3.0e38
IBIG = 2**31 - 1


def _recall_kernel(q_ref, mem_ref, act_ref, out_ref, scores_s, sims_s):
    i = pl.program_id(0)
    q = q_ref[...]
    qn = q / (jnp.sqrt(jnp.sum(q * q)) + 1e-8)
    x = mem_ref[...]
    ones = jnp.ones((1, D), jnp.float32)
    sumsq = jax.lax.dot_general(
        ones, x * x, (((1,), (1,)), ((), ())),
        preferred_element_type=jnp.float32)
    dotq = jax.lax.dot_general(
        qn, x, (((1,), (1,)), ((), ())),
        preferred_element_type=jnp.float32)
    sims = dotq / (jnp.sqrt(sumsq) + 1e-8)
    act = act_ref[0]
    scores = jnp.where(sims > 0.0, act, NEG_BIG)
    sc2 = scores.reshape(SUB, LN)
    sm2 = sims.reshape(SUB, LN)
    sub_iota = jax.lax.broadcasted_iota(jnp.int32, (SUB, LN), 0)
    lane_iota2 = jax.lax.broadcasted_iota(jnp.int32, (SUB, LN), 1)
    g2 = i * BLKP + sub_iota * LN + lane_iota2
    sc2 = jnp.where(g2 < M, sc2, NEG_BIG)
    scores_s[i] = sc2
    sims_s[i] = sm2

    @pl.when(i == NB - 1)
    def _select():
        rm = jnp.max(scores_s[...], axis=2)
        riota = (jax.lax.broadcasted_iota(jnp.int32, (NB, SUB), 0) * SUB
                 + jax.lax.broadcasted_iota(jnp.int32, (NB, SUB), 1))
        lane = jax.lax.broadcasted_iota(jnp.int32, (1, LN), 1)
        lanek = jax.lax.broadcasted_iota(jnp.int32, (1, K), 1)
        out0 = jnp.zeros((1, K), jnp.float32)
        out1 = jnp.zeros((1, K), jnp.float32)
        for k in range(K):
            m = jnp.max(rm)
            sidx = jnp.min(jnp.where(rm == m, riota, IBIG))
            ci = sidx // SUB
            si = sidx % SUB
            prow = scores_s[ci, pl.ds(si, 1), :]
            l = jnp.min(jnp.where(prow == m, lane, IBIG))
            srow = sims_s[ci, pl.ds(si, 1), :]
            simv = jnp.max(jnp.where(lane == l, srow, NEG_INF))
            prow2 = jnp.where(lane == l, NEG_INF, prow)
            scores_s[ci, pl.ds(si, 1), :] = prow2
            rm = jnp.where(riota == sidx, jnp.max(prow2), rm)
            out0 = jnp.where(lanek == k, m, out0)
            out1 = jnp.where(lanek == k, simv, out1)
        out_ref[0:1, :] = out0
        out_ref[1:2, :] = out1


def kernel(query_features, mem_features, activation_strength, topk):
    q = query_features.reshape(1, D)
    act = jnp.pad(activation_strength, (0, NB * BLKP - M)).reshape(NB, 1, BLKP)
    out = pl.pallas_call(
        _recall_kernel,
        grid=(NB,),
        in_specs=[
            pl.BlockSpec((1, D), lambda i: (0, 0)),
            pl.BlockSpec((BLKP, D), lambda i: (i, 0)),
            pl.BlockSpec((1, 1, BLKP), lambda i: (i, 0, 0)),
        ],
        out_specs=pl.BlockSpec((2, K), lambda i: (0, 0)),
        out_shape=jax.ShapeDtypeStruct((2, K), jnp.float32),
        scratch_shapes=[
            pltpu.VMEM((NB, SUB, LN), jnp.float32),
            pltpu.VMEM((NB, SUB, LN), jnp.float32),
        ],
        compiler_params=pltpu.CompilerParams(
            dimension_semantics=("arbitrary",)),
    )(q, mem_features, act)
    toff = (jnp.asarray(topk) - K).astype(jnp.float32)
    return out.at[0, :].add(toff)
